# Initial kernel scaffold; baseline (speedup 1.0000x reference)
import jax
import jax.numpy as jnp
from jax.experimental import pallas as pl


def kernel(x, edge_index, edge_attr, u, batch, W1, b1, W2, b2):
    raise NotImplementedError("write your pallas kernel here")


# speedup vs baseline: 2.6833x; 2.6833x over previous
import functools

import jax
import jax.numpy as jnp
from jax import lax
from jax.experimental import pallas as pl
from jax.experimental.pallas import tpu as pltpu
from jax.experimental.pallas import tpu_sc as plsc

NC = 2
NS = 16
NW = NC * NS
CH = 80


def _mesh():
    return plsc.VectorSubcoreMesh(
        core_axis_name="c", subcore_axis_name="s", num_cores=NC,
        num_subcores=NS)


def _zero_spmem(s, zb_hbm, acc_sh, n):
    n_zchunks = n // CH
    def z_loop(j, carry):
        k = s + j * NS
        @pl.when(k < n_zchunks)
        def _():
            pltpu.sync_copy(zb_hbm, acc_sh.at[pl.ds(k * CH, CH)])
        return carry
    lax.fori_loop(0, (n_zchunks + NS - 1) // NS, z_loop, 0)


def _writeback(s, c, acc_sh, out_hbm, n):
    n_zchunks = n // CH
    def wb_loop(j, carry):
        k = s + j * NS
        @pl.when(k < n_zchunks)
        def _():
            pltpu.sync_copy(acc_sh.at[pl.ds(k * CH, CH)],
                            out_hbm.at[pl.ds(c * n + k * CH, CH)])
        return carry
    lax.fori_loop(0, (n_zchunks + NS - 1) // NS, wb_loop, 0)


def _sc_gather_pass(x, row, col, n, e, d_node):
    n_chunks = e // (NW * CH)
    assert e == n_chunks * NW * CH and n % CH == 0

    @functools.partial(
        pl.kernel,
        out_type=jax.ShapeDtypeStruct((NC * n, d_node), jnp.float32),
        mesh=_mesh(),
        scratch_types=[
            pltpu.VMEM((CH,), jnp.int32),
            pltpu.VMEM((CH,), jnp.int32),
            pltpu.VMEM((CH, d_node), jnp.float32),
            pltpu.VMEM_SHARED((n, d_node), jnp.float32),
            pltpu.SemaphoreType.DMA,
        ],
    )
    def g_kernel(x_hbm, row_hbm, col_hbm, zb_hbm, g_out,
                 idx_v, cidx_v, rows_v, g_sh, sem):
        c = lax.axis_index("c")
        s = lax.axis_index("s")
        _zero_spmem(s, zb_hbm, g_sh, n)
        plsc.subcore_barrier()

        ebase = (c * NS + s) * (n_chunks * CH)
        def body(i, carry):
            base = ebase + i * CH
            pltpu.sync_copy(row_hbm.at[pl.ds(base, CH)], idx_v)
            pltpu.sync_copy(col_hbm.at[pl.ds(base, CH)], cidx_v)
            pltpu.async_copy(x_hbm.at[idx_v], rows_v, sem).wait()
            pltpu.sync_copy(rows_v, g_sh.at[cidx_v], add=True)
            return carry
        lax.fori_loop(0, n_chunks, body, 0)

        plsc.subcore_barrier()
        _writeback(s, c, g_sh, g_out, n)

    zb = jnp.zeros((CH, d_node), jnp.float32)
    return g_kernel(x, row, col, zb)


def _sc_edge_pass(col, ea, n, e, d_node, d_edge):
    n_chunks = e // (NW * CH)

    @functools.partial(
        pl.kernel,
        out_type=jax.ShapeDtypeStruct((NC * n, d_node), jnp.float32),
        mesh=_mesh(),
        scratch_types=[
            pltpu.VMEM((CH,), jnp.int32),
            pltpu.VMEM((CH, d_node), jnp.float32),
            pltpu.VMEM_SHARED((n, d_node), jnp.float32),
        ],
    )
    def e_kernel(col_hbm, eap_hbm, zb_hbm, s_out, cidx_v, pad_v, s_sh):
        c = lax.axis_index("c")
        s = lax.axis_index("s")
        _zero_spmem(s, zb_hbm, s_sh, n)
        plsc.subcore_barrier()

        ebase = (c * NS + s) * (n_chunks * CH)
        def body(i, carry):
            base = ebase + i * CH
            pltpu.sync_copy(col_hbm.at[pl.ds(base, CH)], cidx_v)
            pltpu.sync_copy(eap_hbm.at[pl.ds(base, CH)], pad_v)
            pltpu.sync_copy(pad_v, s_sh.at[cidx_v], add=True)
            return carry
        lax.fori_loop(0, n_chunks, body, 0)

        plsc.subcore_barrier()
        _writeback(s, c, s_sh, s_out, n)

    ea_pad = jnp.concatenate(
        [ea, jnp.ones((e, 1), jnp.float32),
         jnp.zeros((e, d_node - d_edge - 1), jnp.float32)], axis=1)
    zb = jnp.zeros((CH, d_node), jnp.float32)
    return e_kernel(col, ea_pad, zb)


def _tc_dense(x, gp, sp, W1, b1, W2, b2, n, d_node, d_edge):
    def body(x_ref, gp_ref, sp_ref, w1_ref, b1_ref, w2_ref, b2_ref,
             o_ref):
        g = gp_ref[:n, :] + gp_ref[n:, :]
        sd = sp_ref[:n, :] + sp_ref[n:, :]
        s_ = sd[:, :d_edge]
        deg = sd[:, d_edge:d_edge + 1]
        W1x = w1_ref[:, :d_node]
        W1e = w1_ref[:, d_node:]
        W2x = w2_ref[:, :d_node]
        W2a = w2_ref[:, d_node:]
        dn = (((1,), (1,)), ((), ()))
        agg = (lax.dot_general(g, W1x, dn, preferred_element_type=jnp.float32)
               + lax.dot_general(s_, W1e, dn, preferred_element_type=jnp.float32)
               + deg * b1_ref[0, :][None, :])
        out = (lax.dot_general(x_ref[...], W2x, dn,
                               preferred_element_type=jnp.float32)
               + lax.dot_general(agg, W2a, dn,
                                 preferred_element_type=jnp.float32)
               + b2_ref[0, :][None, :])
        o_ref[...] = out

    return pl.pallas_call(
        body,
        out_shape=jax.ShapeDtypeStruct((n, d_node), jnp.float32),
    )(x, gp, sp, W1, b1.reshape(1, -1), W2, b2.reshape(1, -1))


def kernel(x, edge_index, edge_attr, u, batch, W1, b1, W2, b2):
    n, d_node = x.shape
    e, d_edge = edge_attr.shape
    row = edge_index[0]
    col = edge_index[1]
    gp = _sc_gather_pass(x, row, col, n, e, d_node)
    sp = _sc_edge_pass(col, edge_attr, n, e, d_node, d_edge)
    return _tc_dense(x, gp, sp, W1, b1, W2, b2, n, d_node, d_edge)


# speedup vs baseline: 4.0833x; 1.5218x over previous
import functools

import jax
import jax.numpy as jnp
from jax import lax
from jax.experimental import pallas as pl
from jax.experimental.pallas import tpu as pltpu
from jax.experimental.pallas import tpu_sc as plsc

NC = 2
NS = 16
NW = NC * NS
CH = 80
NB = 2


def _mesh():
    return plsc.VectorSubcoreMesh(
        core_axis_name="c", subcore_axis_name="s", num_cores=NC,
        num_subcores=NS)


def _zero_spmem(s, zb_hbm, acc_sh, n, sem):
    n_zchunks = n // CH
    n_iter = (n_zchunks + NS - 1) // NS
    def fire(j, carry):
        k = s + j * NS
        @pl.when(k < n_zchunks)
        def _():
            pltpu.async_copy(zb_hbm, acc_sh.at[pl.ds(k * CH, CH)], sem)
        return carry
    lax.fori_loop(0, n_iter, fire, 0)
    def drain(j, carry):
        k = s + j * NS
        @pl.when(k < n_zchunks)
        def _():
            pltpu.make_async_copy(zb_hbm, acc_sh.at[pl.ds(k * CH, CH)],
                                  sem).wait()
        return carry
    lax.fori_loop(0, n_iter, drain, 0)


def _writeback(s, c, acc_sh, out_hbm, n, sem):
    n_zchunks = n // CH
    n_iter = (n_zchunks + NS - 1) // NS
    def fire(j, carry):
        k = s + j * NS
        @pl.when(k < n_zchunks)
        def _():
            pltpu.async_copy(acc_sh.at[pl.ds(k * CH, CH)],
                             out_hbm.at[pl.ds(c * n + k * CH, CH)], sem)
        return carry
    lax.fori_loop(0, n_iter, fire, 0)
    def drain(j, carry):
        k = s + j * NS
        @pl.when(k < n_zchunks)
        def _():
            pltpu.make_async_copy(acc_sh.at[pl.ds(k * CH, CH)],
                                  out_hbm.at[pl.ds(c * n + k * CH, CH)],
                                  sem).wait()
        return carry
    lax.fori_loop(0, n_iter, drain, 0)


def _sc_gather_pass(x, row4, col4, n, e, d_node):
    n_chunks = e // (NW * CH)
    assert e == n_chunks * NW * CH and n % CH == 0 and n_chunks > NB

    @functools.partial(
        pl.kernel,
        out_type=jax.ShapeDtypeStruct((NC * n, d_node), jnp.float32),
        mesh=_mesh(),
        scratch_types=[
            pltpu.VMEM((NB, 1, CH), jnp.int32),
            pltpu.VMEM((n_chunks, 1, CH), jnp.int32),
            pltpu.VMEM((NB, CH, d_node), jnp.float32),
            pltpu.VMEM_SHARED((n, d_node), jnp.float32),
            pltpu.SemaphoreType.DMA,
            pltpu.SemaphoreType.DMA,
            pltpu.SemaphoreType.DMA,
            pltpu.SemaphoreType.DMA,
            pltpu.SemaphoreType.DMA,
            pltpu.SemaphoreType.DMA,
        ],
    )
    def g_kernel(x_hbm, row_hbm, col_hbm, zb_hbm, g_out,
                 idxr_v, cidx2_v, rows_v, g_sh,
                 sem_g, sem_aux, si0, si1, ss0, ss1):
        c = lax.axis_index("c")
        s = lax.axis_index("s")
        sem_i = [si0, si1]
        sems = [ss0, ss1]
        wid = c * NS + s
        last = n_chunks - 1

        pltpu.async_copy(col_hbm.at[wid], cidx2_v, sem_g)
        pltpu.async_copy(row_hbm.at[wid, 0], idxr_v.at[0], sem_i[0])
        _zero_spmem(s, zb_hbm, g_sh, n, sem_aux)
        pltpu.make_async_copy(col_hbm.at[wid], cidx2_v, sem_g).wait()
        plsc.subcore_barrier()

        def pair(k, carry):
            for b in range(NB):
                i = NB * k + b
                rb = rows_v.at[b]
                civ = cidx2_v.at[i, 0]
                @pl.when(k > 0)
                def _(rb=rb, civ=civ, b=b):
                    pltpu.make_async_copy(rb, g_sh.at[civ], sems[b]).wait()
                pltpu.make_async_copy(row_hbm.at[wid, i], idxr_v.at[b],
                                      sem_i[b]).wait()
                pltpu.async_copy(x_hbm.at[idxr_v.at[b, 0]], rb, sem_g)
                nxt = jnp.minimum(i + 1, last)
                pltpu.async_copy(row_hbm.at[wid, nxt],
                                 idxr_v.at[1 - b], sem_i[1 - b])
                pltpu.make_async_copy(x_hbm.at[idxr_v.at[b, 0]], rb,
                                      sem_g).wait()
                pltpu.async_copy(rb, g_sh.at[civ], sems[b], add=True)
            return carry
        nq = n_chunks // NB
        lax.fori_loop(0, nq, pair, 0)
        if n_chunks % NB:
            i = n_chunks - 1
            b = i % NB
            rb = rows_v.at[b]
            civ = cidx2_v.at[i, 0]
            pltpu.make_async_copy(rb, g_sh.at[civ], sems[b]).wait()
            pltpu.make_async_copy(row_hbm.at[wid, i], idxr_v.at[b],
                                  sem_i[b]).wait()
            pltpu.async_copy(x_hbm.at[idxr_v.at[b, 0]], rb, sem_g).wait()
            pltpu.async_copy(rb, g_sh.at[civ], sems[b], add=True)
        else:
            pltpu.make_async_copy(row_hbm.at[wid, 0], idxr_v.at[0],
                                  sem_i[0]).wait()
        for b in range(NB):
            pltpu.make_async_copy(rows_v.at[b], g_sh.at[cidx2_v.at[b, 0]],
                                  sems[b]).wait()

        plsc.subcore_barrier()
        _writeback(s, c, g_sh, g_out, n, sem_aux)

    zb = jnp.zeros((CH, d_node), jnp.float32)
    return g_kernel(x, row4, col4, zb)


def _sc_edge_pass(col4, ea_pad, n, e, d_node):
    n_chunks = e // (NW * CH)

    @functools.partial(
        pl.kernel,
        out_type=jax.ShapeDtypeStruct((NC * n, d_node), jnp.float32),
        mesh=_mesh(),
        scratch_types=[
            pltpu.VMEM((n_chunks, 1, CH), jnp.int32),
            pltpu.VMEM((NB, CH, d_node), jnp.float32),
            pltpu.VMEM_SHARED((n, d_node), jnp.float32),
            pltpu.SemaphoreType.DMA,
            pltpu.SemaphoreType.DMA,
            pltpu.SemaphoreType.DMA,
            pltpu.SemaphoreType.DMA,
        ],
    )
    def e_kernel(col_hbm, eap_hbm, zb_hbm, s_out,
                 cidx2_v, pad_v, s_sh,
                 sem_l, sem_aux, ss0, ss1):
        c = lax.axis_index("c")
        s = lax.axis_index("s")
        sems = [ss0, ss1]
        wid = c * NS + s
        ebase = wid * n_chunks

        pltpu.async_copy(col_hbm.at[wid], cidx2_v, sem_l)
        _zero_spmem(s, zb_hbm, s_sh, n, sem_aux)
        pltpu.make_async_copy(col_hbm.at[wid], cidx2_v, sem_l).wait()
        plsc.subcore_barrier()

        def pair(k, carry):
            for b in range(NB):
                i = NB * k + b
                rb = pad_v.at[b]
                civ = cidx2_v.at[i, 0]
                @pl.when(k > 0)
                def _(rb=rb, civ=civ, b=b):
                    pltpu.make_async_copy(rb, s_sh.at[civ], sems[b]).wait()
                pltpu.async_copy(eap_hbm.at[pl.ds((ebase + i) * CH, CH)],
                                 rb, sem_l).wait()
                pltpu.async_copy(rb, s_sh.at[civ], sems[b], add=True)
            return carry
        nq = n_chunks // NB
        lax.fori_loop(0, nq, pair, 0)
        for i in range(nq * NB, n_chunks):
            b = i % NB
            rb = pad_v.at[b]
            civ = cidx2_v.at[i, 0]
            pltpu.make_async_copy(rb, s_sh.at[civ], sems[b]).wait()
            pltpu.async_copy(eap_hbm.at[pl.ds((ebase + i) * CH, CH)],
                             rb, sem_l).wait()
            pltpu.async_copy(rb, s_sh.at[civ], sems[b], add=True)
        for b in range(NB):
            pltpu.make_async_copy(pad_v.at[b], s_sh.at[cidx2_v.at[b, 0]],
                                  sems[b]).wait()

        plsc.subcore_barrier()
        _writeback(s, c, s_sh, s_out, n, sem_aux)

    zb = jnp.zeros((CH, d_node), jnp.float32)
    return e_kernel(col4, ea_pad, zb)


def _tc_dense(x, gp, sp, W1, b1, W2, b2, n, d_node, d_edge):
    def body(x_ref, gp_ref, sp_ref, w1_ref, b1_ref, w2_ref, b2_ref,
             o_ref):
        g = gp_ref[:n, :] + gp_ref[n:, :]
        sd = sp_ref[:n, :] + sp_ref[n:, :]
        s_ = sd[:, :d_edge]
        deg = sd[:, d_edge:d_edge + 1]
        W1x = w1_ref[:, :d_node]
        W1e = w1_ref[:, d_node:]
        W2x = w2_ref[:, :d_node]
        W2a = w2_ref[:, d_node:]
        dn = (((1,), (1,)), ((), ()))
        agg = (lax.dot_general(g, W1x, dn, preferred_element_type=jnp.float32)
               + lax.dot_general(s_, W1e, dn, preferred_element_type=jnp.float32)
               + deg * b1_ref[0, :][None, :])
        out = (lax.dot_general(x_ref[...], W2x, dn,
                               preferred_element_type=jnp.float32)
               + lax.dot_general(agg, W2a, dn,
                                 preferred_element_type=jnp.float32)
               + b2_ref[0, :][None, :])
        o_ref[...] = out

    return pl.pallas_call(
        body,
        out_shape=jax.ShapeDtypeStruct((n, d_node), jnp.float32),
    )(x, gp, sp, W1, b1.reshape(1, -1), W2, b2.reshape(1, -1))


def kernel(x, edge_index, edge_attr, u, batch, W1, b1, W2, b2):
    n, d_node = x.shape
    e, d_edge = edge_attr.shape
    n_chunks = e // (NW * CH)
    row4 = edge_index[0].reshape(NW, n_chunks, 1, CH)
    col4 = edge_index[1].reshape(NW, n_chunks, 1, CH)
    ea_pad = jnp.concatenate(
        [edge_attr, jnp.ones((e, 1), jnp.float32),
         jnp.zeros((e, d_node - d_edge - 1), jnp.float32)], axis=1)
    gp = _sc_gather_pass(x, row4, col4, n, e, d_node)
    sp = _sc_edge_pass(col4, ea_pad, n, e, d_node)
    return _tc_dense(x, gp, sp, W1, b1, W2, b2, n, d_node, d_edge)
